# baseline (speedup 1.0000x reference)
import functools

import jax
import jax.numpy as jnp
from jax import lax
from jax.experimental import pallas as pl
from jax.experimental.pallas import tpu as pltpu
from jax.experimental.pallas import tpu_sc as plsc

_B, _S, _D = 16, 2048, 512
_HALF = _S // 2
_CHUNK = 120
_NCH = 9
_TAIL = _HALF - 8 * _CHUNK
_NV = _S // 16

_mesh = plsc.VectorSubcoreMesh(core_axis_name="c", subcore_axis_name="s")


@functools.partial(
    pl.kernel,
    out_type=jax.ShapeDtypeStruct((_B * _S, _D), jnp.float32),
    mesh=_mesh,
    scratch_types=[
        pltpu.VMEM((_S,), jnp.int32),
        pltpu.VMEM((_S,), jnp.int32),
        pltpu.SMEM((_NV,), jnp.int32),
        pltpu.VMEM((_NCH, _CHUNK), jnp.int32),
        pltpu.VMEM((_CHUNK, _D), jnp.float32),
        pltpu.VMEM((_CHUNK, _D), jnp.float32),
        pltpu.SemaphoreType.DMA,
        pltpu.SemaphoreType.DMA,
        pltpu.SemaphoreType.DMA,
        pltpu.SemaphoreType.DMA,
    ],
    compiler_params=pltpu.CompilerParams(needs_layout_passes=False),
)
def _partition_kernel(table_hbm, nl_hbm, out_hbm, keys_v, pa_v, base_v, idx_v,
                      buf0_v, buf1_v, gsem0, gsem1, wsem0, wsem1):
    b = lax.axis_index("s")
    h = lax.axis_index("c")

    pltpu.sync_copy(nl_hbm.at[b], keys_v)

    zeros16 = lax.iota(jnp.int32, 16) * 0
    for o in (_TAIL, _TAIL + 16, _TAIL + 32, _TAIL + 40):
        plsc.store_scatter(
            idx_v,
            [zeros16 + (_NCH - 1), o + lax.iota(jnp.int32, 16)],
            zeros16,
        )

    def scan_body(i, carry):
        for j in range(4):
            o = (i * 4 + j) * 16
            a = jnp.where(keys_v[pl.ds(o, 16)] != 0, 1, 0).astype(jnp.int32)
            pa_v[pl.ds(o, 16)] = plsc.cumsum(a)
        return carry

    lax.fori_loop(0, _NV // 4, scan_body, jnp.int32(0))

    def base_body(i, carry):
        t = pa_v[pl.ds(i * 16, 16)][15]
        base_v[i] = carry
        return carry + t

    a_total = lax.fori_loop(0, _NV, base_body, jnp.int32(0))

    def scatter_body(i, carry):
        for j in range(4):
            v = i * 4 + j
            o = v * 16
            i_vec = o + lax.iota(jnp.int32, 16)
            a = keys_v[pl.ds(o, 16)]
            pa = base_v[v] + pa_v[pl.ds(o, 16)]
            dest = jnp.where(a != 0, pa - 1, a_total + i_vec - pa)
            local = dest - h * _HALF
            mask = (local >= 0) & (local < _HALF)
            safe = jnp.where(mask, local, 0)
            plsc.store_scatter(
                idx_v,
                [safe // _CHUNK, safe % _CHUNK],
                b * _S + i_vec,
                mask=mask,
            )
        return carry

    lax.fori_loop(0, _NV // 4, scatter_body, jnp.int32(0))

    out_base = b * _S + h * _HALF
    bufs = (buf0_v, buf1_v)
    gsems = (gsem0, gsem1)
    wsems = (wsem0, wsem1)
    gather = [None] * _NCH
    write = [None] * _NCH
    for c in range(_NCH):
        p = c % 2
        if c >= 2:
            write[c - 2].wait()
        gather[c] = pltpu.async_copy(table_hbm.at[idx_v.at[c]], bufs[p], gsems[p])
        if c >= 1:
            q = (c - 1) % 2
            gather[c - 1].wait()
            write[c - 1] = pltpu.async_copy(
                bufs[q],
                out_hbm.at[pl.ds(out_base + (c - 1) * _CHUNK, _CHUNK)],
                wsems[q],
            )
    last = _NCH - 1
    gather[last].wait()
    write[last] = pltpu.async_copy(
        bufs[last % 2].at[pl.ds(0, _TAIL)],
        out_hbm.at[pl.ds(out_base + last * _CHUNK, _TAIL)],
        wsems[last % 2],
    )
    write[last - 1].wait()
    write[last].wait()


def kernel(sentout, nl_input):
    table = sentout.reshape(_B * _S, _D)
    nl = nl_input.astype(jnp.int32)
    out = _partition_kernel(table, nl)
    return out.reshape(_B, _S, _D)


# speedup vs baseline: 2.1939x; 2.1939x over previous
import functools

import jax
import jax.numpy as jnp
from jax import lax
from jax.experimental import pallas as pl
from jax.experimental.pallas import tpu as pltpu
from jax.experimental.pallas import tpu_sc as plsc

_B, _S, _D = 16, 2048, 512
_HALF = _S // 2
_CHUNK = 64
_NCH = _HALF // _CHUNK
_NV = _S // 16

_mesh = plsc.VectorSubcoreMesh(core_axis_name="c", subcore_axis_name="s")


@functools.partial(
    pl.kernel,
    out_type=jax.ShapeDtypeStruct((_B * _S, _D), jnp.float32),
    mesh=_mesh,
    scratch_types=[
        pltpu.VMEM((_S,), jnp.int32),
        pltpu.VMEM((_S,), jnp.int32),
        pltpu.SMEM((_NV,), jnp.int32),
        pltpu.VMEM((_NCH, _CHUNK), jnp.int32),
        pltpu.VMEM((_CHUNK, _D), jnp.float32),
        pltpu.VMEM((_CHUNK, _D), jnp.float32),
        pltpu.VMEM((_CHUNK, _D), jnp.float32),
        pltpu.SemaphoreType.DMA,
        pltpu.SemaphoreType.DMA,
        pltpu.SemaphoreType.DMA,
        pltpu.SemaphoreType.DMA,
        pltpu.SemaphoreType.DMA,
        pltpu.SemaphoreType.DMA,
    ],
    compiler_params=pltpu.CompilerParams(needs_layout_passes=False),
)
def _partition_kernel(table_hbm, nl_hbm, out_hbm, keys_v, pa_v, base_v, idx_v,
                      buf0_v, buf1_v, buf2_v,
                      gsem0, gsem1, gsem2, wsem0, wsem1, wsem2):
    b = lax.axis_index("s")
    h = lax.axis_index("c")

    pltpu.sync_copy(nl_hbm.at[b], keys_v)

    def scan_body(i, carry):
        for j in range(4):
            o = (i * 4 + j) * 16
            a = jnp.where(keys_v[pl.ds(o, 16)] != 0, 1, 0).astype(jnp.int32)
            pa_v[pl.ds(o, 16)] = plsc.cumsum(a)
        return carry

    lax.fori_loop(0, _NV // 4, scan_body, jnp.int32(0))

    def base_body(i, carry):
        t = pa_v[pl.ds(i * 16, 16)][15]
        base_v[i] = carry
        return carry + t

    a_total = lax.fori_loop(0, _NV, base_body, jnp.int32(0))

    def scatter_body(i, carry):
        for j in range(4):
            v = i * 4 + j
            o = v * 16
            i_vec = o + lax.iota(jnp.int32, 16)
            a = keys_v[pl.ds(o, 16)]
            pa = base_v[v] + pa_v[pl.ds(o, 16)]
            dest = jnp.where(a != 0, pa - 1, a_total + i_vec - pa)
            local = dest - h * _HALF
            mask = (local >= 0) & (local < _HALF)
            safe = jnp.where(mask, local, 0)
            plsc.store_scatter(
                idx_v,
                [safe >> 6, safe & (_CHUNK - 1)],
                b * _S + i_vec,
                mask=mask,
            )
        return carry

    lax.fori_loop(0, _NV // 4, scatter_body, jnp.int32(0))

    out_base = b * _S + h * _HALF
    bufs = (buf0_v, buf1_v, buf2_v)
    gsems = (gsem0, gsem1, gsem2)
    wsems = (wsem0, wsem1, wsem2)
    nbuf = 3
    gather = [None] * _NCH
    write = [None] * _NCH
    for c in range(_NCH):
        p = c % nbuf
        if c >= nbuf:
            write[c - nbuf].wait()
        gather[c] = pltpu.async_copy(table_hbm.at[idx_v.at[c]], bufs[p], gsems[p])
        if c >= 2:
            q = (c - 2) % nbuf
            gather[c - 2].wait()
            write[c - 2] = pltpu.async_copy(
                bufs[q],
                out_hbm.at[pl.ds(out_base + (c - 2) * _CHUNK, _CHUNK)],
                wsems[q],
            )
    for c in (_NCH - 2, _NCH - 1):
        gather[c].wait()
        write[c] = pltpu.async_copy(
            bufs[c % nbuf],
            out_hbm.at[pl.ds(out_base + c * _CHUNK, _CHUNK)],
            wsems[c % nbuf],
        )
    for c in range(_NCH - nbuf, _NCH):
        write[c].wait()


def kernel(sentout, nl_input):
    table = sentout.reshape(_B * _S, _D)
    nl = nl_input.astype(jnp.int32)
    out = _partition_kernel(table, nl)
    return out.reshape(_B, _S, _D)
